# baseline (speedup 1.0000x reference)
import functools

import jax
import jax.numpy as jnp
from jax import lax
from jax.experimental import pallas as pl
from jax.experimental.pallas import tpu as pltpu
from jax.experimental.pallas import tpu_sc as plsc


def _make_sc_add(B, S, D):
    info = plsc.get_sparse_core_info()
    NC, NS, L = info.num_cores, info.num_subcores, info.num_lanes
    NW = NC * NS
    rows_per_w = S // NW
    CS = 8
    SPAN = CS * D
    NR = 2
    n_chunks = rows_per_w // CS
    n_vec = SPAN // L
    UNROLL = 4

    mesh = plsc.VectorSubcoreMesh(core_axis_name="c", subcore_axis_name="s")

    @functools.partial(
        pl.kernel,
        mesh=mesh,
        out_type=jax.ShapeDtypeStruct((B, S * D), jnp.float32),
        scratch_types=[
            pltpu.VMEM((NR, B, SPAN), jnp.float32),
            pltpu.VMEM((2, SPAN), jnp.float32),
        ]
        + [pltpu.SemaphoreType.DMA] * (2 * NR * B + 2),
    )
    def sc_add(x_hbm, pos_hbm, out_hbm, xbuf, posbuf, *sems):
        ld_sems = sems[:NR * B]
        st_sems = sems[NR * B:2 * NR * B]
        pos_sems = sems[2 * NR * B:]

        wid = lax.axis_index("s") * NC + lax.axis_index("c")
        base = wid * (rows_per_w * D)

        def off0(c):
            return base + c * SPAN

        def add_chunk(slot, pslot):
            def body(iv, carry):
                for u in range(UNROLL):
                    o = iv * (UNROLL * L) + u * L
                    p = posbuf[pslot, pl.ds(o, L)]
                    for b in range(B):
                        xbuf[slot, b, pl.ds(o, L)] = (
                            xbuf[slot, b, pl.ds(o, L)] + p
                        )
                return carry

            lax.fori_loop(0, n_vec // UNROLL, body, None)

        pos_cp = [None, None]
        for c in range(min(2, n_chunks)):
            pos_cp[c] = pltpu.async_copy(
                pos_hbm.at[pl.ds(off0(c), SPAN)], posbuf.at[c], pos_sems[c]
            )
        load_cp = [[None] * B for _ in range(n_chunks)]
        store_cp = [[None] * B for _ in range(n_chunks)]

        for i in range(n_chunks + 1):
            if i < n_chunks:
                slot = i % NR
                for b in range(B):
                    if i >= NR:
                        store_cp[i - NR][b].wait()
                    load_cp[i][b] = pltpu.async_copy(
                        x_hbm.at[b, pl.ds(off0(i), SPAN)],
                        xbuf.at[slot, b],
                        ld_sems[slot * B + b],
                    )
            if i >= 1:
                j = i - 1
                slot = j % NR
                for b in range(B):
                    load_cp[j][b].wait()
                pos_cp[j % 2].wait()
                add_chunk(slot, j % 2)
                if j + 2 < n_chunks:
                    nxt = (j + 2) % 2
                    pos_cp[nxt] = pltpu.async_copy(
                        pos_hbm.at[pl.ds(off0(j + 2), SPAN)],
                        posbuf.at[nxt],
                        pos_sems[nxt],
                    )
                for b in range(B):
                    store_cp[j][b] = pltpu.async_copy(
                        xbuf.at[slot, b],
                        out_hbm.at[b, pl.ds(off0(j), SPAN)],
                        st_sems[slot * B + b],
                    )

        for j in range(max(0, n_chunks - NR), n_chunks):
            for b in range(B):
                store_cp[j][b].wait()

    return sc_add


def kernel(x, pos_table):
    B, S, D = x.shape
    out_flat = _make_sc_add(B, S, D)(
        x.reshape(B, S * D), pos_table.reshape(-1)
    )
    return out_flat.reshape(B, S, D)


# speedup vs baseline: 3.8578x; 3.8578x over previous
import functools

import jax
import jax.numpy as jnp
from jax import lax
from jax.experimental import pallas as pl
from jax.experimental.pallas import tpu as pltpu
from jax.experimental.pallas import tpu_sc as plsc


def _make_sc_add(B, S, D):
    info = plsc.get_sparse_core_info()
    NC, NS, L = info.num_cores, info.num_subcores, info.num_lanes
    NW = NC * NS
    rows_per_w = S // NW
    CS = 8
    NR = 2
    n_chunks = rows_per_w // CS
    vecs_per_row = D // L

    mesh = plsc.VectorSubcoreMesh(core_axis_name="c", subcore_axis_name="s")

    @functools.partial(
        pl.kernel,
        mesh=mesh,
        out_type=jax.ShapeDtypeStruct((B, S, D), jnp.float32),
        scratch_types=[
            pltpu.VMEM((NR, B, CS, D), jnp.float32),
            pltpu.VMEM((2, CS, D), jnp.float32),
        ]
        + [pltpu.SemaphoreType.DMA] * (2 * NR * B + 2),
    )
    def sc_add(x_hbm, pos_hbm, out_hbm, xbuf, posbuf, *sems):
        ld_sems = sems[:NR * B]
        st_sems = sems[NR * B:2 * NR * B]
        pos_sems = sems[2 * NR * B:]

        wid = lax.axis_index("s") * NC + lax.axis_index("c")
        base = wid * rows_per_w

        def row0(c):
            return base + c * CS

        def add_chunk(slot, pslot):
            def body(iv, carry):
                col = iv * L
                for r in range(CS):
                    p = posbuf[pslot, r, pl.ds(col, L)]
                    for b in range(B):
                        xbuf[slot, b, r, pl.ds(col, L)] = (
                            xbuf[slot, b, r, pl.ds(col, L)] + p
                        )
                return carry

            lax.fori_loop(0, vecs_per_row, body, None)

        pos_cp = [None, None]
        for c in range(min(2, n_chunks)):
            pos_cp[c] = pltpu.async_copy(
                pos_hbm.at[pl.ds(row0(c), CS)], posbuf.at[c], pos_sems[c]
            )
        load_cp = [[None] * B for _ in range(n_chunks)]
        store_cp = [[None] * B for _ in range(n_chunks)]

        for i in range(n_chunks + 1):
            if i < n_chunks:
                slot = i % NR
                for b in range(B):
                    if i >= NR:
                        store_cp[i - NR][b].wait()
                    load_cp[i][b] = pltpu.async_copy(
                        x_hbm.at[b, pl.ds(row0(i), CS)],
                        xbuf.at[slot, b],
                        ld_sems[slot * B + b],
                    )
            if i >= 1:
                j = i - 1
                slot = j % NR
                for b in range(B):
                    load_cp[j][b].wait()
                pos_cp[j % 2].wait()
                add_chunk(slot, j % 2)
                if j + 2 < n_chunks:
                    nxt = (j + 2) % 2
                    pos_cp[nxt] = pltpu.async_copy(
                        pos_hbm.at[pl.ds(row0(j + 2), CS)],
                        posbuf.at[nxt],
                        pos_sems[nxt],
                    )
                for b in range(B):
                    store_cp[j][b] = pltpu.async_copy(
                        xbuf.at[slot, b],
                        out_hbm.at[b, pl.ds(row0(j), CS)],
                        st_sems[slot * B + b],
                    )

        for j in range(max(0, n_chunks - NR), n_chunks):
            for b in range(B):
                store_cp[j][b].wait()

    return sc_add


def kernel(x, pos_table):
    B, S, D = x.shape
    return _make_sc_add(B, S, D)(x, pos_table)
